# baseline (device time: 123759 ns/iter reference)
import jax
import jax.numpy as jnp
from jax import lax
from jax.experimental import pallas as pl
from jax.experimental.pallas import tpu as pltpu

P = 8
CPB = 2
NSTEP = P * CPB


def kernel(x, w_mat):
    m_per, k = x.shape
    n = w_mat.shape[1]
    n_per = n // P
    nc = n_per // CPB
    m_tot = m_per * P

    x = x.astype(jnp.bfloat16)

    def body(x_ref, w_ref, out_ref, w_stage, w_bf, send_buf,
             w_sems, loc_sems, send_sems, recv_sems):
        my = lax.axis_index("i")
        idx = pl.program_id(0)

        def dest(i):
            return lax.rem(my + 1 + lax.div(i, CPB), P)

        def chunk(i):
            return lax.rem(i, CPB)

        def w_fetch(i, slot):
            col = dest(i) * n_per + chunk(i) * nc
            return pltpu.make_async_copy(
                w_ref.at[:, pl.ds(col, nc)], w_stage.at[slot], w_sems.at[slot],
            )

        bar = pltpu.get_barrier_semaphore()

        @pl.when(idx == 0)
        def _():
            w_fetch(idx, 0).start()
            w_fetch(idx + 1, 1).start()
            for d in range(P):
                @pl.when(d != my)
                def _():
                    pl.semaphore_signal(
                        bar, inc=1,
                        device_id=(d,), device_id_type=pl.DeviceIdType.MESH,
                    )
            w_fetch(idx, 0).wait()
            w_bf[0, :, :] = w_stage[0].astype(jnp.bfloat16)
            pl.semaphore_wait(bar, P - 1)

        slot = lax.rem(idx, 2)
        nslot = 1 - slot

        @pl.when(idx < NSTEP - 2)
        def _():
            w_fetch(idx + 2, slot).start()

        @pl.when(idx < NSTEP - 1)
        def _():
            w_fetch(idx + 1, nslot).wait()
            w_bf[nslot, :, :] = w_stage[nslot].astype(jnp.bfloat16)

        part = jnp.dot(
            x_ref[:, :], w_bf[slot], preferred_element_type=jnp.float32,
        )
        part = jnp.maximum(part, 0.0).astype(jnp.bfloat16)
        send_buf[idx, :, :] = part

        jj = dest(idx)
        cc = chunk(idx)

        @pl.when(jj != my)
        def _():
            rdma = pltpu.make_async_remote_copy(
                src_ref=send_buf.at[idx],
                dst_ref=out_ref.at[pl.ds(my * m_per, m_per), pl.ds(cc * nc, nc)],
                send_sem=send_sems.at[idx],
                recv_sem=recv_sems.at[my * CPB + cc],
                device_id=(jj,),
                device_id_type=pl.DeviceIdType.MESH,
            )
            rdma.start()

        @pl.when(idx == NSTEP - 1)
        def _():
            for c in range(CPB):
                pltpu.make_async_copy(
                    send_buf.at[NSTEP - CPB + c],
                    out_ref.at[pl.ds(my * m_per, m_per), c * nc:(c + 1) * nc],
                    loc_sems.at[c],
                ).start()

            for src in range(P):
                for c in range(CPB):
                    recv = pltpu.make_async_remote_copy(
                        src_ref=send_buf.at[src * CPB + c],
                        dst_ref=out_ref.at[
                            src * m_per:(src + 1) * m_per, c * nc:(c + 1) * nc
                        ],
                        send_sem=send_sems.at[src * CPB + c],
                        recv_sem=recv_sems.at[src * CPB + c],
                        device_id=(src,),
                        device_id_type=pl.DeviceIdType.MESH,
                    )

                    @pl.when(src != my)
                    def _():
                        recv.wait_recv()

            for i in range(NSTEP - CPB):
                send = pltpu.make_async_remote_copy(
                    src_ref=send_buf.at[i],
                    dst_ref=out_ref.at[0:m_per, 0:nc],
                    send_sem=send_sems.at[i],
                    recv_sem=recv_sems.at[0],
                    device_id=(0,),
                    device_id_type=pl.DeviceIdType.MESH,
                )
                send.wait_send()

            for c in range(CPB):
                pltpu.make_async_copy(
                    send_buf.at[NSTEP - CPB + c],
                    out_ref.at[pl.ds(my * m_per, m_per), c * nc:(c + 1) * nc],
                    loc_sems.at[c],
                ).wait()

    return pl.pallas_call(
        body,
        grid=(NSTEP,),
        out_shape=jax.ShapeDtypeStruct((m_tot, n_per), jnp.bfloat16),
        in_specs=[
            pl.BlockSpec(memory_space=pltpu.VMEM),
            pl.BlockSpec(memory_space=pl.ANY),
        ],
        out_specs=pl.BlockSpec(memory_space=pltpu.VMEM),
        scratch_shapes=[
            pltpu.VMEM((2, k, nc), jnp.float32),
            pltpu.VMEM((2, k, nc), jnp.bfloat16),
            pltpu.VMEM((NSTEP, m_per, nc), jnp.bfloat16),
            pltpu.SemaphoreType.DMA((2,)),
            pltpu.SemaphoreType.DMA((CPB,)),
            pltpu.SemaphoreType.DMA((NSTEP,)),
            pltpu.SemaphoreType.DMA((NSTEP,)),
        ],
        compiler_params=pltpu.CompilerParams(
            collective_id=0,
            vmem_limit_bytes=60 * 1024 * 1024,
            dimension_semantics=("arbitrary",),
        ),
    )(x, w_mat)


# device time: 105645 ns/iter; 1.1715x vs baseline; 1.1715x over previous
import jax
import jax.numpy as jnp
from jax import lax
from jax.experimental import pallas as pl
from jax.experimental.pallas import tpu as pltpu

P = 8
CPB = 2
NSTEP = P * CPB


def kernel(x, w_mat):
    m_per, k = x.shape
    n = w_mat.shape[1]
    n_per = n // P
    nc = n_per // CPB
    m_tot = m_per * P

    xr = 128
    nxc = m_per // xr

    def body(x_ref, w_ref, out_ref, x_bf, x_stage, w_stage, send_buf,
             x_sems, w_sems, loc_sems, send_sems, recv_sems):
        my = lax.axis_index("i")
        idx = pl.program_id(0)

        def dest(i):
            return lax.rem(my + 1 + lax.div(i, CPB), P)

        def chunk(i):
            return lax.rem(i, CPB)

        def w_fetch(i, slot):
            col = dest(i) * n_per + chunk(i) * nc
            return pltpu.make_async_copy(
                w_ref.at[:, pl.ds(col, nc)], w_stage.at[slot], w_sems.at[slot],
            )

        bar = pltpu.get_barrier_semaphore()

        def x_fetch(r, slot):
            return pltpu.make_async_copy(
                x_ref.at[pl.ds(r * xr, xr), :], x_stage.at[slot],
                x_sems.at[slot],
            )

        @pl.when(idx == 0)
        def _():
            w_fetch(idx, 0).start()
            x_fetch(0, 0).start()
            x_fetch(1, 1).start()
            for d in range(P):
                @pl.when(d != my)
                def _():
                    pl.semaphore_signal(
                        bar, inc=1,
                        device_id=(d,), device_id_type=pl.DeviceIdType.MESH,
                    )
            for r in range(nxc):
                x_fetch(r, r % 2).wait()
                x_bf[r * xr:(r + 1) * xr, :] = x_stage[r % 2].astype(
                    jnp.bfloat16
                )
                if r + 2 < nxc:
                    x_fetch(r + 2, r % 2).start()
            pl.semaphore_wait(bar, P - 1)

        slot = lax.rem(idx, 2)

        @pl.when(idx < NSTEP - 1)
        def _():
            w_fetch(idx + 1, 1 - slot).start()

        w_fetch(idx, slot).wait()

        part = jnp.dot(
            x_bf[:, :], w_stage[slot].astype(jnp.bfloat16),
            preferred_element_type=jnp.float32,
        )
        part = jnp.maximum(part, 0.0).astype(jnp.bfloat16)
        send_buf[idx, :, :] = part

        jj = dest(idx)
        cc = chunk(idx)

        @pl.when(jj != my)
        def _():
            rdma = pltpu.make_async_remote_copy(
                src_ref=send_buf.at[idx],
                dst_ref=out_ref.at[pl.ds(my * m_per, m_per), pl.ds(cc * nc, nc)],
                send_sem=send_sems.at[idx],
                recv_sem=recv_sems.at[my * CPB + cc],
                device_id=(jj,),
                device_id_type=pl.DeviceIdType.MESH,
            )
            rdma.start()

        @pl.when(idx == NSTEP - 1)
        def _():
            for c in range(CPB):
                pltpu.make_async_copy(
                    send_buf.at[NSTEP - CPB + c],
                    out_ref.at[pl.ds(my * m_per, m_per), c * nc:(c + 1) * nc],
                    loc_sems.at[c],
                ).start()

            for src in range(P):
                for c in range(CPB):
                    recv = pltpu.make_async_remote_copy(
                        src_ref=send_buf.at[src * CPB + c],
                        dst_ref=out_ref.at[
                            src * m_per:(src + 1) * m_per, c * nc:(c + 1) * nc
                        ],
                        send_sem=send_sems.at[src * CPB + c],
                        recv_sem=recv_sems.at[src * CPB + c],
                        device_id=(src,),
                        device_id_type=pl.DeviceIdType.MESH,
                    )

                    @pl.when(src != my)
                    def _():
                        recv.wait_recv()

            for i in range(NSTEP - CPB):
                send = pltpu.make_async_remote_copy(
                    src_ref=send_buf.at[i],
                    dst_ref=out_ref.at[0:m_per, 0:nc],
                    send_sem=send_sems.at[i],
                    recv_sem=recv_sems.at[0],
                    device_id=(0,),
                    device_id_type=pl.DeviceIdType.MESH,
                )
                send.wait_send()

            for c in range(CPB):
                pltpu.make_async_copy(
                    send_buf.at[NSTEP - CPB + c],
                    out_ref.at[pl.ds(my * m_per, m_per), c * nc:(c + 1) * nc],
                    loc_sems.at[c],
                ).wait()

    return pl.pallas_call(
        body,
        grid=(NSTEP,),
        out_shape=jax.ShapeDtypeStruct((m_tot, n_per), jnp.bfloat16),
        in_specs=[
            pl.BlockSpec(memory_space=pl.ANY),
            pl.BlockSpec(memory_space=pl.ANY),
        ],
        out_specs=pl.BlockSpec(memory_space=pltpu.VMEM),
        scratch_shapes=[
            pltpu.VMEM((m_per, k), jnp.bfloat16),
            pltpu.VMEM((2, xr, k), jnp.float32),
            pltpu.VMEM((2, k, nc), jnp.float32),
            pltpu.VMEM((NSTEP, m_per, nc), jnp.bfloat16),
            pltpu.SemaphoreType.DMA((2,)),
            pltpu.SemaphoreType.DMA((2,)),
            pltpu.SemaphoreType.DMA((CPB,)),
            pltpu.SemaphoreType.DMA((NSTEP,)),
            pltpu.SemaphoreType.DMA((NSTEP,)),
        ],
        compiler_params=pltpu.CompilerParams(
            collective_id=0,
            vmem_limit_bytes=58 * 1024 * 1024,
            dimension_semantics=("arbitrary",),
        ),
    )(x, w_mat)
